# baseline (speedup 1.0000x reference)
import functools

import jax
import jax.numpy as jnp
import numpy as np
from jax import lax
from jax.experimental import pallas as pl
from jax.experimental.pallas import tpu as pltpu
from jax.experimental.pallas import tpu_sc as plsc


@functools.lru_cache(maxsize=None)
def _pe_const(seq_len: int, d_model: int):
    pos = np.arange(seq_len, dtype=np.float32)[:, None]
    div = np.exp(
        np.arange(0, d_model, 2, dtype=np.float32) * (-np.log(10000.0) / d_model)
    )
    ang = pos * div[None, :]
    pe = np.zeros((seq_len, d_model), dtype=np.float32)
    pe[:, 0::2] = np.sin(ang)
    pe[:, 1::2] = np.cos(ang)
    return pe


@functools.lru_cache(maxsize=None)
def _build_sc_kernel(B: int, S: int, V: int, D: int):
    info = plsc.get_sparse_core_info()
    NC, NS, L = info.num_cores, info.num_subcores, info.num_lanes
    NW = NC * NS
    assert D % L == 0
    assert S % NW == 0
    s_per_w = S // NW
    CH = s_per_w // 2
    n_chunks = B * 2

    mesh = plsc.VectorSubcoreMesh(core_axis_name="c", subcore_axis_name="s")

    @functools.partial(
        pl.kernel,
        mesh=mesh,
        out_type=jax.ShapeDtypeStruct((B, S, D), jnp.float32),
        scratch_types=[
            pltpu.VMEM((B * s_per_w,), jnp.int32),
            pltpu.VMEM((s_per_w, D), jnp.float32),
            pltpu.VMEM((2, CH, D), jnp.float32),
            pltpu.SemaphoreType.DMA,
            pltpu.SemaphoreType.DMA,
            pltpu.SemaphoreType.DMA,
            pltpu.SemaphoreType.DMA,
        ],
    )
    def k(x_hbm, table_hbm, pe_hbm, out_hbm, idx_v, pe_v, rows_v,
          sem0, sem1, sem2, sem3):
        sems = (sem0, sem1)
        st_sems = (sem2, sem3)
        wid = lax.axis_index("s") * NC + lax.axis_index("c")
        base_s = wid * s_per_w

        for b in range(B):
            pltpu.sync_copy(
                x_hbm.at[pl.ds(b * S + base_s, s_per_w)],
                idx_v.at[pl.ds(b * s_per_w, s_per_w)],
            )
        pltpu.sync_copy(pe_hbm.at[pl.ds(base_s, s_per_w)], pe_v)

        def start_gather(c, buf):
            return pltpu.async_copy(
                table_hbm.at[idx_v.at[pl.ds(c * CH, CH)]],
                rows_v.at[buf],
                sems[buf],
            )

        copies = [None] * n_chunks
        stores = [None] * n_chunks
        copies[0] = start_gather(0, 0)
        for c in range(n_chunks):
            cur = c % 2
            copies[c].wait()
            if c + 1 < n_chunks:
                if c >= 1:
                    stores[c - 1].wait()
                copies[c + 1] = start_gather(c + 1, 1 - cur)
            b, h = divmod(c, 2)

            def row_body(r, _, cur=cur, h=h):
                for j in range(D // L):
                    sl = pl.ds(j * L, L)
                    plsc.addupdate(rows_v.at[cur, r, sl], pe_v[h * CH + r, sl])
                return 0

            lax.fori_loop(0, CH, row_body, 0)
            stores[c] = pltpu.async_copy(
                rows_v.at[cur],
                out_hbm.at[b, pl.ds(base_s + h * CH, CH)],
                st_sems[cur],
            )
        stores[n_chunks - 1].wait()

    return k


def kernel(x, table):
    B, S = x.shape
    V, D = table.shape
    pe = jnp.asarray(_pe_const(S, D))
    k = _build_sc_kernel(B, S, V, D)
    return k(x.reshape(B * S).astype(jnp.int32), table, pe)


# speedup vs baseline: 1.2198x; 1.2198x over previous
import functools

import jax
import jax.numpy as jnp
import numpy as np
from jax import lax
from jax.experimental import pallas as pl
from jax.experimental.pallas import tpu as pltpu
from jax.experimental.pallas import tpu_sc as plsc


@functools.lru_cache(maxsize=None)
def _pe_const(seq_len: int, d_model: int):
    pos = np.arange(seq_len, dtype=np.float32)[:, None]
    div = np.exp(
        np.arange(0, d_model, 2, dtype=np.float32) * (-np.log(10000.0) / d_model)
    )
    ang = pos * div[None, :]
    pe = np.zeros((seq_len, d_model), dtype=np.float32)
    pe[:, 0::2] = np.sin(ang)
    pe[:, 1::2] = np.cos(ang)
    return pe


@functools.lru_cache(maxsize=None)
def _build_sc_kernel(B: int, S: int, V: int, D: int):
    info = plsc.get_sparse_core_info()
    NC, NS, L = info.num_cores, info.num_subcores, info.num_lanes
    NW = NC * NS
    assert D % L == 0
    assert S % NW == 0
    s_per_w = S // NW
    CH = s_per_w // 2
    n_chunks = B * 2

    mesh = plsc.VectorSubcoreMesh(core_axis_name="c", subcore_axis_name="s")

    @functools.partial(
        pl.kernel,
        mesh=mesh,
        out_type=jax.ShapeDtypeStruct((B, S, D), jnp.float32),
        scratch_types=[
            pltpu.VMEM((B * s_per_w,), jnp.int32),
            pltpu.VMEM((s_per_w, D), jnp.float32),
            pltpu.VMEM((3, CH, D), jnp.float32),
            pltpu.SemaphoreType.DMA,
            pltpu.SemaphoreType.DMA,
            pltpu.SemaphoreType.DMA,
            pltpu.SemaphoreType.DMA,
            pltpu.SemaphoreType.DMA,
            pltpu.SemaphoreType.DMA,
        ],
    )
    def k(x_hbm, table_hbm, pe_hbm, out_hbm, idx_v, pe_v, rows_v,
          sem0, sem1, sem2, sem3, sem4, sem5):
        sems = (sem0, sem1, sem2)
        st_sems = (sem3, sem4, sem5)
        wid = lax.axis_index("s") * NC + lax.axis_index("c")
        base_s = wid * s_per_w

        for b in range(B):
            pltpu.sync_copy(
                x_hbm.at[pl.ds(b * S + base_s, s_per_w)],
                idx_v.at[pl.ds(b * s_per_w, s_per_w)],
            )
        pltpu.sync_copy(pe_hbm.at[pl.ds(base_s, s_per_w)], pe_v)

        def start_gather(c, buf):
            return pltpu.async_copy(
                table_hbm.at[idx_v.at[pl.ds(c * CH, CH)]],
                rows_v.at[buf],
                sems[buf],
            )

        copies = [None] * n_chunks
        stores = [None] * n_chunks
        copies[0] = start_gather(0, 0)
        copies[1] = start_gather(1, 1)
        for c in range(n_chunks):
            cur = c % 3
            copies[c].wait()
            b, h = divmod(c, 2)

            @plsc.parallel_loop(0, CH, step=1)
            def row_body(r):
                for j in range(D // L):
                    sl = pl.ds(j * L, L)
                    plsc.addupdate(rows_v.at[cur, r, sl], pe_v[h * CH + r, sl])

            stores[c] = pltpu.async_copy(
                rows_v.at[cur],
                out_hbm.at[b, pl.ds(base_s + h * CH, CH)],
                st_sems[cur],
            )
            if c + 2 < n_chunks:
                if c >= 1:
                    stores[c - 1].wait()
                copies[c + 2] = start_gather(c + 2, (c + 2) % 3)
        stores[n_chunks - 2].wait()
        stores[n_chunks - 1].wait()

    return k


def kernel(x, table):
    B, S = x.shape
    V, D = table.shape
    pe = jnp.asarray(_pe_const(S, D))
    k = _build_sc_kernel(B, S, V, D)
    return k(x.reshape(B * S).astype(jnp.int32), table, pe)


# speedup vs baseline: 1.2706x; 1.0416x over previous
import functools

import jax
import jax.numpy as jnp
import numpy as np
from jax import lax
from jax.experimental import pallas as pl
from jax.experimental.pallas import tpu as pltpu
from jax.experimental.pallas import tpu_sc as plsc


@functools.lru_cache(maxsize=None)
def _pe_const(seq_len: int, d_model: int):
    pos = np.arange(seq_len, dtype=np.float32)[:, None]
    div = np.exp(
        np.arange(0, d_model, 2, dtype=np.float32) * (-np.log(10000.0) / d_model)
    )
    ang = pos * div[None, :]
    pe = np.zeros((seq_len, d_model), dtype=np.float32)
    pe[:, 0::2] = np.sin(ang)
    pe[:, 1::2] = np.cos(ang)
    return pe


@functools.lru_cache(maxsize=None)
def _build_sc_kernel(B: int, S: int, V: int, D: int):
    info = plsc.get_sparse_core_info()
    NC, NS, L = info.num_cores, info.num_subcores, info.num_lanes
    NW = NC * NS
    assert D % L == 0
    assert S % NW == 0
    s_per_w = S // NW
    CH = s_per_w // 2
    n_chunks = B * 2

    mesh = plsc.VectorSubcoreMesh(core_axis_name="c", subcore_axis_name="s")

    @functools.partial(
        pl.kernel,
        mesh=mesh,
        out_type=jax.ShapeDtypeStruct((B, S, D), jnp.float32),
        scratch_types=[
            pltpu.VMEM((B * s_per_w,), jnp.int32),
            pltpu.VMEM((s_per_w, D), jnp.float32),
            pltpu.VMEM((3, CH, D), jnp.float32),
            pltpu.SemaphoreType.DMA,
            pltpu.SemaphoreType.DMA,
            pltpu.SemaphoreType.DMA,
            pltpu.SemaphoreType.DMA,
            pltpu.SemaphoreType.DMA,
            pltpu.SemaphoreType.DMA,
            pltpu.SemaphoreType.DMA,
        ],
    )
    def k(x_hbm, table_hbm, pe_hbm, out_hbm, idx_v, pe_v, rows_v,
          sem0, sem1, sem2, sem3, sem4, sem5, pe_sem):
        sems = (sem0, sem1, sem2)
        st_sems = (sem3, sem4, sem5)
        wid = lax.axis_index("s") * NC + lax.axis_index("c")
        base_s = wid * s_per_w

        pe_copy = pltpu.async_copy(
            pe_hbm.at[pl.ds(base_s, s_per_w)], pe_v, pe_sem
        )
        for b in range(B):
            pltpu.sync_copy(
                x_hbm.at[pl.ds(b * S + base_s, s_per_w)],
                idx_v.at[pl.ds(b * s_per_w, s_per_w)],
            )

        def start_gather(c, buf):
            return pltpu.async_copy(
                table_hbm.at[idx_v.at[pl.ds(c * CH, CH)]],
                rows_v.at[buf],
                sems[buf],
            )

        copies = [None] * n_chunks
        stores = [None] * n_chunks
        copies[0] = start_gather(0, 0)
        copies[1] = start_gather(1, 1)
        pe_copy.wait()
        for c in range(n_chunks):
            cur = c % 3
            copies[c].wait()
            b, h = divmod(c, 2)

            @plsc.parallel_loop(0, CH, step=1)
            def row_body(r):
                for j in range(D // L):
                    sl = pl.ds(j * L, L)
                    plsc.addupdate(rows_v.at[cur, r, sl], pe_v[h * CH + r, sl])

            stores[c] = pltpu.async_copy(
                rows_v.at[cur],
                out_hbm.at[b, pl.ds(base_s + h * CH, CH)],
                st_sems[cur],
            )
            if c + 2 < n_chunks:
                if c >= 1:
                    stores[c - 1].wait()
                copies[c + 2] = start_gather(c + 2, (c + 2) % 3)
        stores[n_chunks - 2].wait()
        stores[n_chunks - 1].wait()

    return k


def kernel(x, table):
    B, S = x.shape
    V, D = table.shape
    pe = jnp.asarray(_pe_const(S, D))
    k = _build_sc_kernel(B, S, V, D)
    return k(x.reshape(B * S).astype(jnp.int32), table, pe)


# speedup vs baseline: 1.2805x; 1.0078x over previous
import functools

import jax
import jax.numpy as jnp
import numpy as np
from jax import lax
from jax.experimental import pallas as pl
from jax.experimental.pallas import tpu as pltpu
from jax.experimental.pallas import tpu_sc as plsc


@functools.lru_cache(maxsize=None)
def _pe_const(seq_len: int, d_model: int):
    pos = np.arange(seq_len, dtype=np.float32)[:, None]
    div = np.exp(
        np.arange(0, d_model, 2, dtype=np.float32) * (-np.log(10000.0) / d_model)
    )
    ang = pos * div[None, :]
    pe = np.zeros((seq_len, d_model), dtype=np.float32)
    pe[:, 0::2] = np.sin(ang)
    pe[:, 1::2] = np.cos(ang)
    return pe


@functools.lru_cache(maxsize=None)
def _build_sc_kernel(B: int, S: int, V: int, D: int):
    info = plsc.get_sparse_core_info()
    NC, NS, L = info.num_cores, info.num_subcores, info.num_lanes
    NW = NC * NS
    assert D % L == 0
    assert S % NW == 0
    s_per_w = S // NW
    CH = s_per_w // 2
    n_chunks = B * 2

    mesh = plsc.VectorSubcoreMesh(core_axis_name="c", subcore_axis_name="s")

    @functools.partial(
        pl.kernel,
        mesh=mesh,
        out_type=jax.ShapeDtypeStruct((B, S, D), jnp.float32),
        scratch_types=[
            pltpu.VMEM((B * s_per_w,), jnp.int32),
            pltpu.VMEM((s_per_w, D), jnp.float32),
            pltpu.VMEM((3, CH, D), jnp.float32),
            pltpu.SemaphoreType.DMA,
            pltpu.SemaphoreType.DMA,
            pltpu.SemaphoreType.DMA,
            pltpu.SemaphoreType.DMA,
            pltpu.SemaphoreType.DMA,
            pltpu.SemaphoreType.DMA,
            pltpu.SemaphoreType.DMA,
        ],
    )
    def k(x_hbm, table_hbm, pe_hbm, out_hbm, idx_v, pe_v, rows_v,
          sem0, sem1, sem2, sem3, sem4, sem5, pe_sem):
        sems = (sem0, sem1, sem2)
        st_sems = (sem3, sem4, sem5)
        wid = lax.axis_index("s") * NC + lax.axis_index("c")
        base_s = wid * s_per_w

        pe_copy = pltpu.async_copy(
            pe_hbm.at[pl.ds(base_s, s_per_w)], pe_v, pe_sem
        )
        idx_copies = [
            pltpu.async_copy(
                x_hbm.at[b, pl.ds(base_s, s_per_w)],
                idx_v.at[pl.ds(b * s_per_w, s_per_w)],
                sems[2],
            )
            for b in range(B)
        ]
        for cp in idx_copies:
            cp.wait()

        def start_gather(c, buf):
            return pltpu.async_copy(
                table_hbm.at[idx_v.at[pl.ds(c * CH, CH)]],
                rows_v.at[buf],
                sems[buf],
            )

        copies = [None] * n_chunks
        stores = [None] * n_chunks
        copies[0] = start_gather(0, 0)
        copies[1] = start_gather(1, 1)
        pe_copy.wait()
        for c in range(n_chunks):
            cur = c % 3
            copies[c].wait()
            b, h = divmod(c, 2)

            @plsc.parallel_loop(0, CH, step=1)
            def row_body(r):
                for j in range(D // L):
                    sl = pl.ds(j * L, L)
                    plsc.addupdate(rows_v.at[cur, r, sl], pe_v[h * CH + r, sl])

            stores[c] = pltpu.async_copy(
                rows_v.at[cur],
                out_hbm.at[b, pl.ds(base_s + h * CH, CH)],
                st_sems[cur],
            )
            if c + 2 < n_chunks:
                if c >= 1:
                    stores[c - 1].wait()
                copies[c + 2] = start_gather(c + 2, (c + 2) % 3)
        stores[n_chunks - 2].wait()
        stores[n_chunks - 1].wait()

    return k


def kernel(x, table):
    B, S = x.shape
    V, D = table.shape
    pe = jnp.asarray(_pe_const(S, D))
    k = _build_sc_kernel(B, S, V, D)
    return k(x.astype(jnp.int32), table, pe)
